# baseline (speedup 1.0000x reference)
import functools

import jax
import jax.numpy as jnp
from jax import lax
from jax.experimental import pallas as pl
from jax.experimental.pallas import tpu as pltpu
from jax.experimental.pallas import tpu_sc as plsc

B, H, D = 16384, 200, 32
V = 1000000
NC, NS = 2, 16
NW = NC * NS
PER_W = B // NW
RPC = 4
CHUNK = RPC * H
KR = 10
GL = CHUNK // KR
NCH = PER_W // RPC
L = 16


def _sc_gather_sum(x0, x1, table):
  mesh = plsc.VectorSubcoreMesh(core_axis_name="c", subcore_axis_name="s")

  @functools.partial(
      pl.kernel,
      out_type=jax.ShapeDtypeStruct((B, D), jnp.float32),
      mesh=mesh,
      scratch_types=[
          pltpu.VMEM((3, CHUNK), jnp.int32),
          pltpu.VMEM((2, CHUNK, D), jnp.float32),
          pltpu.VMEM((PER_W, D), jnp.float32),
          pltpu.SemaphoreType.DMA,
          pltpu.SemaphoreType.DMA,
      ],
      compiler_params=pltpu.CompilerParams(use_tc_tiling_on_sc=False),
  )
  def k(x0_hbm, x1_hbm, tab_hbm, out_hbm, idx_v, rows_v, acc_v, isem, gsem):
    cid = lax.axis_index("c")
    sid = lax.axis_index("s")
    wid = cid * NS + sid
    xoff = sid * (NCH * CHUNK)

    def _fetch_idx(ci, slot):
      src = pl.ds(xoff + ci * CHUNK, CHUNK)

      @pl.when(cid == 0)
      def _():
        pltpu.async_copy(x0_hbm.at[src], idx_v.at[slot], isem)

      @pl.when(cid == 1)
      def _():
        pltpu.async_copy(x1_hbm.at[src], idx_v.at[slot], isem)

    _fetch_idx(0, 0)

    zero = jnp.zeros((L,), jnp.float32)

    def _accum(cj, pslot):
      for q in range(RPC):
        a = (zero, zero)

        def _r(rr, acc):
          a0, a1 = acc
          for u in range(10):
            i = q * H + rr * 10 + u
            a0 = a0 + rows_v[pslot, i, pl.ds(0, L)]
            a1 = a1 + rows_v[pslot, i, pl.ds(L, L)]
          return (a0, a1)

        a = lax.fori_loop(0, H // 10, _r, a)
        row = cj * RPC + q
        acc_v[row, pl.ds(0, L)] = a[0]
        acc_v[row, pl.ds(L, L)] = a[1]

    def _chunk(ci, carry):
      islot = lax.rem(ci, 3)
      gslot = lax.rem(ci, 2)
      pslot = lax.rem(ci + 1, 2)
      pltpu.make_async_copy(
          x0_hbm.at[pl.ds(xoff, CHUNK)], idx_v.at[islot], isem).wait()

      for j in range(KR):
        pltpu.async_copy(
            tab_hbm.at[idx_v.at[islot, pl.ds(j * GL, GL)]],
            rows_v.at[gslot, pl.ds(j * GL, GL)], gsem)

      @pl.when(ci + 1 < NCH)
      def _():
        _fetch_idx(ci + 1, lax.rem(ci + 1, 3))

      @pl.when(ci >= 1)
      def _():
        for j in range(KR):
          pltpu.make_async_copy(
              tab_hbm.at[pl.ds(0, GL)],
              rows_v.at[pslot, pl.ds(j * GL, GL)], gsem).wait()
        _accum(ci - 1, pslot)

      return carry

    lax.fori_loop(0, NCH, _chunk, 0)

    lslot = lax.rem(NCH - 1, 2)
    for j in range(KR):
      pltpu.make_async_copy(
          tab_hbm.at[pl.ds(0, GL)],
          rows_v.at[lslot, pl.ds(j * GL, GL)], gsem).wait()
    _accum(NCH - 1, lslot)

    pltpu.sync_copy(acc_v, out_hbm.at[pl.ds(wid * PER_W, PER_W)])

  return k(x0, x1, table)


def _tc_body(s_ref, wt_ref, b_ref, o_ref):
  o_ref[...] = (
      jnp.dot(s_ref[...], wt_ref[...], preferred_element_type=jnp.float32)
      * (1.0 / H)
      + b_ref[...]
  )


def _tc_project(sums, wt, b2):
  blk = 2048
  return pl.pallas_call(
      _tc_body,
      grid=(B // blk,),
      in_specs=[
          pl.BlockSpec((blk, D), lambda i: (i, 0)),
          pl.BlockSpec((D, D), lambda i: (0, 0)),
          pl.BlockSpec((1, D), lambda i: (0, 0)),
      ],
      out_specs=pl.BlockSpec((blk, D), lambda i: (i, 0)),
      out_shape=jax.ShapeDtypeStruct((B, D), jnp.float32),
  )(sums, wt, b2)


def kernel(x, table, W, b):
  x_flat = jnp.minimum(x.astype(jnp.int32), jnp.int32(V - 1)).reshape(B * H)
  half = B * H // 2
  sums = _sc_gather_sum(x_flat[:half], x_flat[half:], table)
  return _tc_project(sums, W.T, b.reshape(1, D))


# speedup vs baseline: 1.0156x; 1.0156x over previous
import functools

import jax
import jax.numpy as jnp
from jax import lax
from jax.experimental import pallas as pl
from jax.experimental.pallas import tpu as pltpu
from jax.experimental.pallas import tpu_sc as plsc

B, H, D = 16384, 200, 32
V = 1000000
NC, NS = 2, 16
NW = NC * NS
PER_W = B // NW
RPC = 4
CHUNK = RPC * H
KR = 10
GL = CHUNK // KR
NCH = PER_W // RPC
SLAB = CHUNK // D
L = 16


def _sc_gather_sum(x_flat, table):
  mesh = plsc.VectorSubcoreMesh(core_axis_name="c", subcore_axis_name="s")

  @functools.partial(
      pl.kernel,
      out_type=jax.ShapeDtypeStruct((B, D), jnp.float32),
      mesh=mesh,
      scratch_types=[
          pltpu.VMEM((3, SLAB, D), jnp.int32),
          pltpu.VMEM((2, CHUNK), jnp.int32),
          pltpu.VMEM((2, CHUNK, D), jnp.float32),
          pltpu.VMEM((PER_W, D), jnp.float32),
          pltpu.SemaphoreType.DMA,
          pltpu.SemaphoreType.DMA,
      ],
      compiler_params=pltpu.CompilerParams(use_tc_tiling_on_sc=False),
  )
  def k(x_hbm, tab_hbm, out_hbm, raw_v, idx_v, rows_v, acc_v, isem, gsem):
    cid = lax.axis_index("c")
    sid = lax.axis_index("s")
    wid = cid * NS + sid
    xrow0 = wid * (NCH * SLAB)

    pltpu.async_copy(x_hbm.at[pl.ds(xrow0, SLAB)], raw_v.at[0], isem)

    zero = jnp.zeros((L,), jnp.float32)

    def _accum(cj, pslot):
      for q in range(RPC):
        a = (zero, zero)

        def _r(rr, acc):
          a0, a1 = acc
          for u in range(10):
            i = q * H + rr * 10 + u
            a0 = a0 + rows_v[pslot, i, pl.ds(0, L)]
            a1 = a1 + rows_v[pslot, i, pl.ds(L, L)]
          return (a0, a1)

        a = lax.fori_loop(0, H // 10, _r, a)
        row = cj * RPC + q
        acc_v[row, pl.ds(0, L)] = a[0]
        acc_v[row, pl.ds(L, L)] = a[1]

    def _chunk(ci, carry):
      islot = lax.rem(ci, 3)
      gslot = lax.rem(ci, 2)
      pslot = lax.rem(ci + 1, 2)
      pltpu.make_async_copy(
          x_hbm.at[pl.ds(xrow0, SLAB)], raw_v.at[islot], isem).wait()

      for rr in range(SLAB):
        idx_v[gslot, pl.ds(rr * D, L)] = raw_v[islot, rr, pl.ds(0, L)]
        idx_v[gslot, pl.ds(rr * D + L, L)] = raw_v[islot, rr, pl.ds(L, L)]

      for j in range(KR):
        pltpu.async_copy(
            tab_hbm.at[idx_v.at[gslot, pl.ds(j * GL, GL)]],
            rows_v.at[gslot, pl.ds(j * GL, GL)], gsem)

      @pl.when(ci + 1 < NCH)
      def _():
        pltpu.async_copy(
            x_hbm.at[pl.ds(xrow0 + (ci + 1) * SLAB, SLAB)],
            raw_v.at[lax.rem(ci + 1, 3)], isem)

      @pl.when(ci >= 1)
      def _():
        for j in range(KR):
          pltpu.make_async_copy(
              tab_hbm.at[pl.ds(0, GL)],
              rows_v.at[pslot, pl.ds(j * GL, GL)], gsem).wait()
        _accum(ci - 1, pslot)

      return carry

    lax.fori_loop(0, NCH, _chunk, 0)

    lslot = lax.rem(NCH - 1, 2)
    for j in range(KR):
      pltpu.make_async_copy(
          tab_hbm.at[pl.ds(0, GL)],
          rows_v.at[lslot, pl.ds(j * GL, GL)], gsem).wait()
    _accum(NCH - 1, lslot)

    pltpu.sync_copy(acc_v, out_hbm.at[pl.ds(wid * PER_W, PER_W)])

  return k(x_flat, table)


def _tc_body(s_ref, wt_ref, b_ref, o_ref):
  o_ref[...] = (
      jnp.dot(s_ref[...], wt_ref[...], preferred_element_type=jnp.float32)
      * (1.0 / H)
      + b_ref[...]
  )


def _tc_project(sums, wt, b2):
  blk = 2048
  return pl.pallas_call(
      _tc_body,
      grid=(B // blk,),
      in_specs=[
          pl.BlockSpec((blk, D), lambda i: (i, 0)),
          pl.BlockSpec((D, D), lambda i: (0, 0)),
          pl.BlockSpec((1, D), lambda i: (0, 0)),
      ],
      out_specs=pl.BlockSpec((blk, D), lambda i: (i, 0)),
      out_shape=jax.ShapeDtypeStruct((B, D), jnp.float32),
  )(sums, wt, b2)


def kernel(x, table, W, b):
  x_rows = jnp.minimum(x.astype(jnp.int32), jnp.int32(V - 1)).reshape(
      B * H // D, D)
  sums = _sc_gather_sum(x_rows, table)
  return _tc_project(sums, W.T, b.reshape(1, D))


# speedup vs baseline: 1.0290x; 1.0131x over previous
import functools

import jax
import jax.numpy as jnp
from jax import lax
from jax.experimental import pallas as pl
from jax.experimental.pallas import tpu as pltpu
from jax.experimental.pallas import tpu_sc as plsc

B, H, D = 16384, 200, 32
V = 1000000
NC, NS = 2, 16
NW = NC * NS
PER_W = B // NW
RPC = 4
CHUNK = RPC * H
KR = 10
GL = CHUNK // KR
NCH = PER_W // RPC
L = 16


def _sc_gather_sum(x_flat, table):
  mesh = plsc.VectorSubcoreMesh(core_axis_name="c", subcore_axis_name="s")

  @functools.partial(
      pl.kernel,
      out_type=jax.ShapeDtypeStruct((B, D), jnp.float32),
      mesh=mesh,
      scratch_types=[
          pltpu.VMEM((3, CHUNK), jnp.int32),
          pltpu.VMEM((2, CHUNK, D), jnp.float32),
          pltpu.VMEM((PER_W, D), jnp.float32),
          pltpu.SemaphoreType.DMA,
          pltpu.SemaphoreType.DMA,
      ],
      compiler_params=pltpu.CompilerParams(use_tc_tiling_on_sc=False),
  )
  def k(tab_hbm, x_hbm, out_hbm, idx_v, rows_v, acc_v, isem, gsem):
    cid = lax.axis_index("c")
    sid = lax.axis_index("s")
    wid = cid * NS + sid
    xoff = wid * (NCH * CHUNK)

    pltpu.async_copy(x_hbm.at[pl.ds(xoff, CHUNK)], idx_v.at[0], isem)

    zero = jnp.zeros((L,), jnp.float32)

    def _accum(cj, pslot):
      for q in range(RPC):
        a = (zero, zero)

        def _r(rr, acc):
          a0, a1 = acc
          for u in range(10):
            i = q * H + rr * 10 + u
            a0 = a0 + rows_v[pslot, i, pl.ds(0, L)]
            a1 = a1 + rows_v[pslot, i, pl.ds(L, L)]
          return (a0, a1)

        a = lax.fori_loop(0, H // 10, _r, a)
        row = cj * RPC + q
        acc_v[row, pl.ds(0, L)] = a[0]
        acc_v[row, pl.ds(L, L)] = a[1]

    def _chunk(ci, carry):
      islot = lax.rem(ci, 3)
      gslot = lax.rem(ci, 2)
      pslot = lax.rem(ci + 1, 2)
      pltpu.make_async_copy(
          x_hbm.at[pl.ds(xoff, CHUNK)], idx_v.at[islot], isem).wait()

      for j in range(KR):
        pltpu.async_copy(
            tab_hbm.at[idx_v.at[islot, pl.ds(j * GL, GL)]],
            rows_v.at[gslot, pl.ds(j * GL, GL)], gsem)

      @pl.when(ci + 1 < NCH)
      def _():
        pltpu.async_copy(
            x_hbm.at[pl.ds(xoff + (ci + 1) * CHUNK, CHUNK)],
            idx_v.at[lax.rem(ci + 1, 3)], isem)

      @pl.when(ci >= 1)
      def _():
        for j in range(KR):
          pltpu.make_async_copy(
              tab_hbm.at[pl.ds(0, GL)],
              rows_v.at[pslot, pl.ds(j * GL, GL)], gsem).wait()
        _accum(ci - 1, pslot)

      return carry

    lax.fori_loop(0, NCH, _chunk, 0)

    lslot = lax.rem(NCH - 1, 2)
    for j in range(KR):
      pltpu.make_async_copy(
          tab_hbm.at[pl.ds(0, GL)],
          rows_v.at[lslot, pl.ds(j * GL, GL)], gsem).wait()
    _accum(NCH - 1, lslot)

    pltpu.sync_copy(acc_v, out_hbm.at[pl.ds(wid * PER_W, PER_W)])

  return k(table, x_flat)


def _tc_body(s_ref, wt_ref, b_ref, o_ref):
  o_ref[...] = (
      jnp.dot(s_ref[...], wt_ref[...], preferred_element_type=jnp.float32)
      * (1.0 / H)
      + b_ref[...]
  )


def _tc_project(sums, wt, b2):
  blk = 2048
  return pl.pallas_call(
      _tc_body,
      grid=(B // blk,),
      in_specs=[
          pl.BlockSpec((blk, D), lambda i: (i, 0)),
          pl.BlockSpec((D, D), lambda i: (0, 0)),
          pl.BlockSpec((1, D), lambda i: (0, 0)),
      ],
      out_specs=pl.BlockSpec((blk, D), lambda i: (i, 0)),
      out_shape=jax.ShapeDtypeStruct((B, D), jnp.float32),
  )(sums, wt, b2)


def kernel(x, table, W, b):
  x_flat = jnp.minimum(x.astype(jnp.int32), jnp.int32(V - 1)).reshape(B * H)
  sums = _sc_gather_sum(x_flat, table)
  return _tc_project(sums, W.T, b.reshape(1, D))
